# Initial kernel scaffold; baseline (speedup 1.0000x reference)
import jax
import jax.numpy as jnp
from jax.experimental import pallas as pl


def kernel(x, edge_index, edge_label_index, W1, b1, W2, b2, W3, b3, g1, be1, g2, be2, dW1, db1, dW2, db2, dW3, db3, dW4, db4):
    raise NotImplementedError("write your pallas kernel here")


# speedup vs baseline: 11.2549x; 11.2549x over previous
import functools

import jax
import jax.numpy as jnp
from jax import lax
from jax.experimental import pallas as pl
from jax.experimental.pallas import tpu as pltpu
from jax.experimental.pallas import tpu_sc as plsc

N = 10000
E = 320000
P = 100000
EPS = 1e-5

NC = 2
NS = 16
NW = NC * NS

NP = 10240
ROWS_PER_TILE = NP // NS

ER = 2560
EPAD = ER * 128
PR = 800
PPAD = PR * 128

@functools.cache
def _get_mesh():
    return plsc.VectorSubcoreMesh(core_axis_name="c", subcore_axis_name="s",
                                  num_cores=NC, num_subcores=NS)


def _sc_degree(dst2, ones_w, zeros_w):
    R = ER // NW

    @functools.partial(
        pl.kernel,
        out_type=jax.ShapeDtypeStruct((NC, NP, 128), jnp.float32),
        mesh=_get_mesh(),
        scratch_types=[
            pltpu.VMEM((R, 128), jnp.int32),
            pltpu.VMEM((128, 128), jnp.float32),
            pltpu.VMEM_SHARED((NP, 128), jnp.float32),
            pltpu.SemaphoreType.DMA,
        ],
    )
    def k(dst_hbm, ones_hbm, zeros_hbm, out_hbm, idx_v, ones_v, acc_sh, sem):
        c = lax.axis_index("c")
        s = lax.axis_index("s")
        wid = c * NS + s
        pltpu.sync_copy(zeros_hbm, acc_sh.at[pl.ds(s * ROWS_PER_TILE, ROWS_PER_TILE)])
        pltpu.sync_copy(ones_hbm, ones_v)
        pltpu.sync_copy(dst_hbm.at[pl.ds(wid * R, R)], idx_v)
        plsc.subcore_barrier()

        def body(j, carry):
            pltpu.sync_copy(ones_v, acc_sh.at[idx_v.at[j]], add=True)
            return carry

        lax.fori_loop(0, R, body, 0)
        plsc.subcore_barrier()
        pltpu.sync_copy(acc_sh.at[pl.ds(s * ROWS_PER_TILE, ROWS_PER_TILE)],
                        out_hbm.at[c, pl.ds(s * ROWS_PER_TILE, ROWS_PER_TILE)])

    return k(dst2, ones_w, zeros_w)


def _sc_aggregate(tab, src2, dst2, zeros_w, hh):
    R = ER // NS
    RH = R // 2

    @functools.partial(
        pl.kernel,
        out_type=jax.ShapeDtypeStruct((NC, NP, hh), jnp.float32),
        mesh=_get_mesh(),
        scratch_types=[
            pltpu.VMEM((RH, 128), jnp.int32),
            pltpu.VMEM((RH, 128), jnp.int32),
            pltpu.VMEM((128, hh), jnp.float32),
            pltpu.VMEM_SHARED((NP, hh), jnp.float32),
            pltpu.SemaphoreType.DMA,
        ],
    )
    def k(ta_hbm, tb_hbm, src_hbm, dst_hbm, zeros_hbm, out_hbm, src_v, dst_v,
          rows_v, acc_sh, sem):
        c = lax.axis_index("c")
        s = lax.axis_index("s")
        pltpu.sync_copy(zeros_hbm, acc_sh.at[pl.ds(s * ROWS_PER_TILE, ROWS_PER_TILE)])
        plsc.subcore_barrier()

        def body(j, carry):
            @pl.when(c == 0)
            def _():
                pltpu.async_copy(ta_hbm.at[src_v.at[j]], rows_v, sem).wait()

            @pl.when(c == 1)
            def _():
                pltpu.async_copy(tb_hbm.at[src_v.at[j]], rows_v, sem).wait()

            pltpu.sync_copy(rows_v, acc_sh.at[dst_v.at[j]], add=True)
            return carry

        for ph in range(2):
            pltpu.sync_copy(src_hbm.at[pl.ds(s * R + ph * RH, RH)], src_v)
            pltpu.sync_copy(dst_hbm.at[pl.ds(s * R + ph * RH, RH)], dst_v)
            lax.fori_loop(0, RH, body, 0)
        plsc.subcore_barrier()
        pltpu.sync_copy(acc_sh.at[pl.ds(s * ROWS_PER_TILE, ROWS_PER_TILE)],
                        out_hbm.at[c, pl.ds(s * ROWS_PER_TILE, ROWS_PER_TILE)])

    return k(tab[0], tab[1], src2, dst2, zeros_w)


def _sc_aggregate_full(tab, src2, dst2, zeros_w):
    R = ER // NW

    @functools.partial(
        pl.kernel,
        out_type=jax.ShapeDtypeStruct((NC, NP, 128), jnp.float32),
        mesh=_get_mesh(),
        scratch_types=[
            pltpu.VMEM((R, 128), jnp.int32),
            pltpu.VMEM((R, 128), jnp.int32),
            pltpu.VMEM((128, 128), jnp.float32),
            pltpu.VMEM_SHARED((NP, 128), jnp.float32),
            pltpu.SemaphoreType.DMA,
        ],
    )
    def k(tab_hbm, src_hbm, dst_hbm, zeros_hbm, out_hbm, src_v, dst_v, rows_v,
          acc_sh, sem):
        c = lax.axis_index("c")
        s = lax.axis_index("s")
        wid = c * NS + s
        pltpu.sync_copy(zeros_hbm, acc_sh.at[pl.ds(s * ROWS_PER_TILE, ROWS_PER_TILE)])
        pltpu.sync_copy(src_hbm.at[pl.ds(wid * R, R)], src_v)
        pltpu.sync_copy(dst_hbm.at[pl.ds(wid * R, R)], dst_v)
        plsc.subcore_barrier()

        def body(j, carry):
            pltpu.async_copy(tab_hbm.at[src_v.at[j]], rows_v, sem).wait()
            pltpu.sync_copy(rows_v, acc_sh.at[dst_v.at[j]], add=True)
            return carry

        lax.fori_loop(0, R, body, 0)
        plsc.subcore_barrier()
        pltpu.sync_copy(acc_sh.at[pl.ds(s * ROWS_PER_TILE, ROWS_PER_TILE)],
                        out_hbm.at[c, pl.ds(s * ROWS_PER_TILE, ROWS_PER_TILE)])

    return k(tab, src2, dst2, zeros_w)


def _sc_pair_gather(z, eli2):
    R = PR // NW

    @functools.partial(
        pl.kernel,
        out_type=[jax.ShapeDtypeStruct((PPAD, 128), jnp.float32)] * 2,
        mesh=_get_mesh(),
        scratch_types=[
            pltpu.VMEM((R, 128), jnp.int32),
            pltpu.VMEM((128, 128), jnp.float32),
            pltpu.SemaphoreType.DMA,
        ],
    )
    def k(z_hbm, eli_hbm, o1_hbm, o2_hbm, idx_v, rows_v, sem):
        c = lax.axis_index("c")
        s = lax.axis_index("s")
        wid = c * NS + s
        for side, o_hbm in ((0, o1_hbm), (1, o2_hbm)):
            pltpu.sync_copy(eli_hbm.at[side, wid], idx_v)

            def body(j, carry):
                pltpu.async_copy(z_hbm.at[idx_v.at[j]], rows_v, sem).wait()
                pltpu.sync_copy(rows_v,
                                o_hbm.at[pl.ds(wid * R * 128 + j * 128, 128)])
                return carry

            lax.fori_loop(0, R, body, 0)

    return k(z, eli2)


def _tc_prep(x, w1, d0, d1):
    def body(x_ref, w_ref, d0_ref, d1_ref, t_ref, dinv_ref):
        deg = d0_ref[...] + d1_ref[...] + 1.0
        dinv = lax.rsqrt(deg)
        dinv_ref[...] = dinv
        t = jnp.dot(x_ref[...], w_ref[...],
                    preferred_element_type=jnp.float32) * dinv
        t_ref[0] = t[:, :128]
        t_ref[1] = t[:, 128:]

    return pl.pallas_call(
        body,
        out_shape=[jax.ShapeDtypeStruct((2, N, 128), jnp.float32),
                   jax.ShapeDtypeStruct((N, 1), jnp.float32)],
    )(x, w1, d0, d1)


def _tc_mid(agg, t, dinv, b2d, g2d, be2d, wn, split_out):
    hn = wn.shape[1]

    def body(agg_ref, t_ref, dinv_ref, b_ref, g_ref, be_ref, w_ref, to_ref):
        dinv = dinv_ref[...]
        hs = []
        for c in range(2):
            h = (agg_ref[c, :N, :] + t_ref[c]) * dinv + b_ref[:, c * 128:(c + 1) * 128]
            m = jnp.mean(h, axis=0, keepdims=True)
            v = jnp.mean((h - m) ** 2, axis=0, keepdims=True)
            h = (h - m) * lax.rsqrt(v + EPS) * g_ref[:, c * 128:(c + 1) * 128] \
                + be_ref[:, c * 128:(c + 1) * 128]
            hs.append(jnp.maximum(h, 0.0))
        tn = (jnp.dot(hs[0], w_ref[:128, :], preferred_element_type=jnp.float32)
              + jnp.dot(hs[1], w_ref[128:, :], preferred_element_type=jnp.float32)
              ) * dinv
        if split_out:
            to_ref[0] = tn[:, :hn // 2]
            to_ref[1] = tn[:, hn // 2:]
        else:
            to_ref[...] = tn

    shape = (2, N, hn // 2) if split_out else (N, hn)
    return pl.pallas_call(
        body,
        out_shape=jax.ShapeDtypeStruct(shape, jnp.float32),
    )(agg, t, dinv, b2d, g2d, be2d, wn)


def _tc_z(agg, t, dinv, b3):
    def body(agg_ref, t_ref, dinv_ref, b_ref, z_ref):
        dinv = dinv_ref[...]
        z_ref[...] = (agg_ref[0, :N, :] + agg_ref[1, :N, :] + t_ref[...]) * dinv \
            + b_ref[...]

    return pl.pallas_call(
        body,
        out_shape=jax.ShapeDtypeStruct((N, 128), jnp.float32),
    )(agg, t, dinv, b3)


def _tc_decoder(d1, d2, w1a, w1b, b1, w2, b2, w3, b3, w4, b4):
    BP = 2048
    grid = PPAD // BP

    def body(d1_ref, d2_ref, w1a_ref, w1b_ref, b1_ref, w2_ref, b2_ref,
             w3_ref, b3_ref, w4_ref, b4_ref, o_ref):
        h = jnp.dot(d1_ref[...], w1a_ref[...], preferred_element_type=jnp.float32) \
            + jnp.dot(d2_ref[...], w1b_ref[...], preferred_element_type=jnp.float32) \
            + b1_ref[...]
        h = jnp.maximum(h, 0.0)
        h = jnp.maximum(jnp.dot(h, w2_ref[...],
                                preferred_element_type=jnp.float32) + b2_ref[...], 0.0)
        h = jnp.maximum(jnp.dot(h, w3_ref[...],
                                preferred_element_type=jnp.float32) + b3_ref[...], 0.0)
        o_ref[...] = jnp.dot(h, w4_ref[...],
                             preferred_element_type=jnp.float32) + b4_ref[...]

    def full(w):
        return pl.BlockSpec(w.shape, lambda i: tuple(0 for _ in w.shape))

    ws = (w1a, w1b, b1, w2, b2, w3, b3, w4, b4)
    return pl.pallas_call(
        body,
        grid=(grid,),
        in_specs=[pl.BlockSpec((BP, 128), lambda i: (i, 0)),
                  pl.BlockSpec((BP, 128), lambda i: (i, 0))] + [full(w) for w in ws],
        out_specs=pl.BlockSpec((BP, 1), lambda i: (i, 0)),
        out_shape=jax.ShapeDtypeStruct((PPAD, 1), jnp.float32),
    )(d1, d2, *ws)


def kernel(x, edge_index, edge_label_index, W1, b1, W2, b2, W3, b3, g1, be1,
           g2, be2, dW1, db1, dW2, db2, dW3, db3, dW4, db4):
    f32 = jnp.float32
    i32 = jnp.int32

    pad_e = EPAD - E
    fill = jnp.arange(pad_e, dtype=i32)
    src_p = jnp.concatenate([edge_index[0], fill % N]).reshape(ER, 128)
    dst_p = jnp.concatenate([edge_index[1], N + fill % (NP - N)]).reshape(ER, 128)

    pad_p = PPAD - P
    pfill = jnp.arange(pad_p, dtype=i32) % N
    eli2 = jnp.concatenate(
        [edge_label_index, jnp.stack([pfill, pfill])],
        axis=1).reshape(2, NW, PR // NW, 128)

    zeros128 = jnp.zeros((ROWS_PER_TILE, 128), f32)
    ones128 = jnp.ones((128, 128), f32)

    degp = _sc_degree(dst_p, ones128, zeros128)
    d0 = degp[0, :N, 0:1]
    d1 = degp[1, :N, 0:1]

    t1, dinv = _tc_prep(x, W1, d0, d1)
    agg1 = _sc_aggregate(t1, src_p, dst_p, zeros128, 128)
    t2 = _tc_mid(agg1, t1, dinv, b1.reshape(1, -1), g1.reshape(1, -1),
                 be1.reshape(1, -1), W2, split_out=True)
    agg2 = _sc_aggregate(t2, src_p, dst_p, zeros128, 128)
    t3 = _tc_mid(agg2, t2, dinv, b2.reshape(1, -1), g2.reshape(1, -1),
                 be2.reshape(1, -1), W3, split_out=False)
    agg3 = _sc_aggregate_full(t3, src_p, dst_p, zeros128)
    z = _tc_z(agg3, t3, dinv, b3.reshape(1, -1))

    dz1, dz2 = _sc_pair_gather(z, eli2)
    logits = _tc_decoder(dz1, dz2, dW1[:128], dW1[128:], db1.reshape(1, -1),
                         dW2, db2.reshape(1, -1), dW3, db3.reshape(1, -1),
                         dW4, db4.reshape(1, -1))
    return logits[:P]


# speedup vs baseline: 13.1228x; 1.1660x over previous
import functools

import jax
import jax.numpy as jnp
from jax import lax
from jax.experimental import pallas as pl
from jax.experimental.pallas import tpu as pltpu
from jax.experimental.pallas import tpu_sc as plsc

N = 10000
E = 320000
P = 100000
EPS = 1e-5

NC = 2
NS = 16
NW = NC * NS

NP = 10240
ROWS_PER_TILE = NP // NS

ER = 2560
EPAD = ER * 128
PR = 800
PPAD = PR * 128

@functools.cache
def _get_mesh():
    return plsc.VectorSubcoreMesh(core_axis_name="c", subcore_axis_name="s",
                                  num_cores=NC, num_subcores=NS)


def _sc_degree(dst2, ones_w, zeros_w):
    R = ER // NW

    @functools.partial(
        pl.kernel,
        out_type=jax.ShapeDtypeStruct((NC, NP, 128), jnp.float32),
        mesh=_get_mesh(),
        scratch_types=[
            pltpu.VMEM((R, 128), jnp.int32),
            pltpu.VMEM((128, 128), jnp.float32),
            pltpu.VMEM_SHARED((NP, 128), jnp.float32),
            pltpu.SemaphoreType.DMA,
        ],
    )
    def k(dst_hbm, ones_hbm, zeros_hbm, out_hbm, idx_v, ones_v, acc_sh, sem):
        c = lax.axis_index("c")
        s = lax.axis_index("s")
        wid = c * NS + s
        pltpu.sync_copy(zeros_hbm, acc_sh.at[pl.ds(s * ROWS_PER_TILE, ROWS_PER_TILE)])
        pltpu.sync_copy(ones_hbm, ones_v)
        pltpu.sync_copy(dst_hbm.at[pl.ds(wid * R, R)], idx_v)
        plsc.subcore_barrier()

        K8 = 8

        def body(cid, carry):
            for u in range(K8):
                pltpu.async_copy(ones_v, acc_sh.at[idx_v.at[cid * K8 + u]], sem,
                                 add=True)
            for u in range(K8):
                pltpu.make_async_copy(ones_v, acc_sh.at[idx_v.at[cid * K8 + u]],
                                      sem).wait()
            return carry

        lax.fori_loop(0, R // K8, body, 0)
        plsc.subcore_barrier()
        pltpu.sync_copy(acc_sh.at[pl.ds(s * ROWS_PER_TILE, ROWS_PER_TILE)],
                        out_hbm.at[c, pl.ds(s * ROWS_PER_TILE, ROWS_PER_TILE)])

    return k(dst2, ones_w, zeros_w)


def _agg_phase_pipeline(tab_hbm, src_v, dst_v, rows_a, rows_b, sem_a, sem_b,
                        acc_sh, nrows):
    pltpu.async_copy(tab_hbm.at[src_v.at[0]], rows_a, sem_a)

    def pair(p, carry):
        j0 = 2 * p
        j1 = j0 + 1
        pltpu.make_async_copy(tab_hbm.at[src_v.at[j0]], rows_a, sem_a).wait()
        pltpu.async_copy(tab_hbm.at[src_v.at[j1]], rows_b, sem_b)
        pltpu.sync_copy(rows_a, acc_sh.at[dst_v.at[j0]], add=True)
        pltpu.make_async_copy(tab_hbm.at[src_v.at[j1]], rows_b, sem_b).wait()

        @pl.when(j1 + 1 < nrows)
        def _():
            pltpu.async_copy(tab_hbm.at[src_v.at[j1 + 1]], rows_a, sem_a)

        pltpu.sync_copy(rows_b, acc_sh.at[dst_v.at[j1]], add=True)
        return carry

    lax.fori_loop(0, nrows // 2, pair, 0)


def _sc_aggregate(tab2n, src_pair, dst2, zeros_w):
    R = ER // NS
    RH = 40

    @functools.partial(
        pl.kernel,
        out_type=jax.ShapeDtypeStruct((NC, NP, 128), jnp.float32),
        mesh=_get_mesh(),
        scratch_types=[
            pltpu.VMEM((RH, 128), jnp.int32),
            pltpu.VMEM((RH, 128), jnp.int32),
            pltpu.VMEM((128, 128), jnp.float32),
            pltpu.VMEM((128, 128), jnp.float32),
            pltpu.VMEM_SHARED((NP, 128), jnp.float32),
            pltpu.SemaphoreType.DMA,
            pltpu.SemaphoreType.DMA,
        ],
    )
    def k(tab_hbm, src_hbm, dst_hbm, zeros_hbm, out_hbm, src_v, dst_v,
          rows_a, rows_b, acc_sh, sem_a, sem_b):
        c = lax.axis_index("c")
        s = lax.axis_index("s")
        pltpu.sync_copy(zeros_hbm, acc_sh.at[pl.ds(s * ROWS_PER_TILE, ROWS_PER_TILE)])
        plsc.subcore_barrier()

        for ph in range(R // RH):
            pltpu.sync_copy(src_hbm.at[c, pl.ds(s * R + ph * RH, RH)], src_v)
            pltpu.sync_copy(dst_hbm.at[pl.ds(s * R + ph * RH, RH)], dst_v)
            _agg_phase_pipeline(tab_hbm, src_v, dst_v, rows_a, rows_b,
                                sem_a, sem_b, acc_sh, RH)
        plsc.subcore_barrier()
        pltpu.sync_copy(acc_sh.at[pl.ds(s * ROWS_PER_TILE, ROWS_PER_TILE)],
                        out_hbm.at[c, pl.ds(s * ROWS_PER_TILE, ROWS_PER_TILE)])

    return k(tab2n, src_pair, dst2, zeros_w)


def _sc_aggregate_full(tab, src2, dst2, zeros_w):
    R = ER // NW
    RH = 40

    @functools.partial(
        pl.kernel,
        out_type=jax.ShapeDtypeStruct((NC, NP, 128), jnp.float32),
        mesh=_get_mesh(),
        scratch_types=[
            pltpu.VMEM((RH, 128), jnp.int32),
            pltpu.VMEM((RH, 128), jnp.int32),
            pltpu.VMEM((128, 128), jnp.float32),
            pltpu.VMEM((128, 128), jnp.float32),
            pltpu.VMEM_SHARED((NP, 128), jnp.float32),
            pltpu.SemaphoreType.DMA,
            pltpu.SemaphoreType.DMA,
        ],
    )
    def k(tab_hbm, src_hbm, dst_hbm, zeros_hbm, out_hbm, src_v, dst_v,
          rows_a, rows_b, acc_sh, sem_a, sem_b):
        c = lax.axis_index("c")
        s = lax.axis_index("s")
        wid = c * NS + s
        pltpu.sync_copy(zeros_hbm, acc_sh.at[pl.ds(s * ROWS_PER_TILE, ROWS_PER_TILE)])
        plsc.subcore_barrier()

        for ph in range(R // RH):
            pltpu.sync_copy(src_hbm.at[pl.ds(wid * R + ph * RH, RH)], src_v)
            pltpu.sync_copy(dst_hbm.at[pl.ds(wid * R + ph * RH, RH)], dst_v)
            _agg_phase_pipeline(tab_hbm, src_v, dst_v, rows_a, rows_b,
                                sem_a, sem_b, acc_sh, RH)
        plsc.subcore_barrier()
        pltpu.sync_copy(acc_sh.at[pl.ds(s * ROWS_PER_TILE, ROWS_PER_TILE)],
                        out_hbm.at[c, pl.ds(s * ROWS_PER_TILE, ROWS_PER_TILE)])

    return k(tab, src2, dst2, zeros_w)


def _sc_pair_gather(z, eli3):
    R = PR // NW
    R2 = 2 * R

    @functools.partial(
        pl.kernel,
        out_type=jax.ShapeDtypeStruct((2, PPAD, 128), jnp.float32),
        mesh=_get_mesh(),
        scratch_types=[
            pltpu.VMEM((R2, 128), jnp.int32),
            pltpu.VMEM((128, 128), jnp.float32),
            pltpu.VMEM((128, 128), jnp.float32),
            pltpu.SemaphoreType.DMA,
            pltpu.SemaphoreType.DMA,
        ],
    )
    def k(z_hbm, eli_hbm, out_hbm, idx_v, rows_a, rows_b, sem_a, sem_b):
        c = lax.axis_index("c")
        s = lax.axis_index("s")
        wid = c * NS + s
        pltpu.sync_copy(eli_hbm.at[wid], idx_v)

        def off(j):
            side = j // R
            return side, wid * R * 128 + (j % R) * 128

        pltpu.async_copy(z_hbm.at[idx_v.at[0]], rows_a, sem_a)

        def pair(p, carry):
            j0 = 2 * p
            j1 = j0 + 1
            pltpu.make_async_copy(z_hbm.at[idx_v.at[j0]], rows_a, sem_a).wait()
            pltpu.async_copy(z_hbm.at[idx_v.at[j1]], rows_b, sem_b)
            s0, o0 = off(j0)
            pltpu.sync_copy(rows_a, out_hbm.at[s0, pl.ds(o0, 128)])
            pltpu.make_async_copy(z_hbm.at[idx_v.at[j1]], rows_b, sem_b).wait()

            @pl.when(j1 + 1 < R2)
            def _():
                pltpu.async_copy(z_hbm.at[idx_v.at[j1 + 1]], rows_a, sem_a)

            s1, o1 = off(j1)
            pltpu.sync_copy(rows_b, out_hbm.at[s1, pl.ds(o1, 128)])
            return carry

        lax.fori_loop(0, R, pair, 0)

    return k(z, eli3)


def _tc_prep(x, w1, d0, d1):
    def body(x_ref, w_ref, d0_ref, d1_ref, t_ref, dinv_ref):
        deg = d0_ref[...] + d1_ref[...] + 1.0
        dinv = lax.rsqrt(deg)
        dinv_ref[...] = dinv
        t = jnp.dot(x_ref[...], w_ref[...],
                    preferred_element_type=jnp.float32) * dinv
        t_ref[0] = t[:, :128]
        t_ref[1] = t[:, 128:]

    return pl.pallas_call(
        body,
        out_shape=[jax.ShapeDtypeStruct((2, N, 128), jnp.float32),
                   jax.ShapeDtypeStruct((N, 1), jnp.float32)],
    )(x, w1, d0, d1)


def _tc_mid(agg, t, dinv, b2d, g2d, be2d, wn, split_out):
    hn = wn.shape[1]

    def body(agg_ref, t_ref, dinv_ref, b_ref, g_ref, be_ref, w_ref, to_ref):
        dinv = dinv_ref[...]
        hs = []
        for c in range(2):
            h = (agg_ref[c, :N, :] + t_ref[c]) * dinv + b_ref[:, c * 128:(c + 1) * 128]
            m = jnp.mean(h, axis=0, keepdims=True)
            v = jnp.mean((h - m) ** 2, axis=0, keepdims=True)
            h = (h - m) * lax.rsqrt(v + EPS) * g_ref[:, c * 128:(c + 1) * 128] \
                + be_ref[:, c * 128:(c + 1) * 128]
            hs.append(jnp.maximum(h, 0.0))
        tn = (jnp.dot(hs[0], w_ref[:128, :], preferred_element_type=jnp.float32)
              + jnp.dot(hs[1], w_ref[128:, :], preferred_element_type=jnp.float32)
              ) * dinv
        if split_out:
            to_ref[0] = tn[:, :hn // 2]
            to_ref[1] = tn[:, hn // 2:]
        else:
            to_ref[...] = tn

    shape = (2, N, hn // 2) if split_out else (N, hn)
    return pl.pallas_call(
        body,
        out_shape=jax.ShapeDtypeStruct(shape, jnp.float32),
    )(agg, t, dinv, b2d, g2d, be2d, wn)


def _tc_z(agg, t, dinv, b3):
    def body(agg_ref, t_ref, dinv_ref, b_ref, z_ref):
        dinv = dinv_ref[...]
        z_ref[...] = (agg_ref[0, :N, :] + agg_ref[1, :N, :] + t_ref[...]) * dinv \
            + b_ref[...]

    return pl.pallas_call(
        body,
        out_shape=jax.ShapeDtypeStruct((N, 128), jnp.float32),
    )(agg, t, dinv, b3)


def _tc_decoder(d1, d2, w1a, w1b, b1, w2, b2, w3, b3, w4, b4):
    BP = 2048
    grid = PPAD // BP

    def body(d1_ref, d2_ref, w1a_ref, w1b_ref, b1_ref, w2_ref, b2_ref,
             w3_ref, b3_ref, w4_ref, b4_ref, o_ref):
        h = jnp.dot(d1_ref[...], w1a_ref[...], preferred_element_type=jnp.float32) \
            + jnp.dot(d2_ref[...], w1b_ref[...], preferred_element_type=jnp.float32) \
            + b1_ref[...]
        h = jnp.maximum(h, 0.0)
        h = jnp.maximum(jnp.dot(h, w2_ref[...],
                                preferred_element_type=jnp.float32) + b2_ref[...], 0.0)
        h = jnp.maximum(jnp.dot(h, w3_ref[...],
                                preferred_element_type=jnp.float32) + b3_ref[...], 0.0)
        o_ref[...] = jnp.dot(h, w4_ref[...],
                             preferred_element_type=jnp.float32) + b4_ref[...]

    def full(w):
        return pl.BlockSpec(w.shape, lambda i: tuple(0 for _ in w.shape))

    ws = (w1a, w1b, b1, w2, b2, w3, b3, w4, b4)
    return pl.pallas_call(
        body,
        grid=(grid,),
        in_specs=[pl.BlockSpec((BP, 128), lambda i: (i, 0)),
                  pl.BlockSpec((BP, 128), lambda i: (i, 0))] + [full(w) for w in ws],
        out_specs=pl.BlockSpec((BP, 1), lambda i: (i, 0)),
        out_shape=jax.ShapeDtypeStruct((PPAD, 1), jnp.float32),
    )(d1, d2, *ws)


def kernel(x, edge_index, edge_label_index, W1, b1, W2, b2, W3, b3, g1, be1,
           g2, be2, dW1, db1, dW2, db2, dW3, db3, dW4, db4):
    f32 = jnp.float32
    i32 = jnp.int32

    pad_e = EPAD - E
    fill = jnp.arange(pad_e, dtype=i32)
    src_flat = jnp.concatenate([edge_index[0], fill % N])
    src_pair = jnp.stack([src_flat, src_flat + N]).reshape(2, ER, 128)
    src_p = src_flat.reshape(ER, 128)
    dst_p = jnp.concatenate([edge_index[1], N + fill % (NP - N)]).reshape(ER, 128)

    pad_p = PPAD - P
    pfill = jnp.arange(pad_p, dtype=i32) % N
    eli3 = jnp.concatenate(
        [edge_label_index, jnp.stack([pfill, pfill])],
        axis=1).reshape(2, NW, PR // NW, 128).transpose(1, 0, 2, 3).reshape(
            NW, 2 * (PR // NW), 128)

    zeros128 = jnp.zeros((ROWS_PER_TILE, 128), f32)
    ones128 = jnp.ones((128, 128), f32)

    degp = _sc_degree(dst_p, ones128, zeros128)
    d0 = degp[0, :N, 0:1]
    d1 = degp[1, :N, 0:1]

    t1, dinv = _tc_prep(x, W1, d0, d1)
    agg1 = _sc_aggregate(t1.reshape(2 * N, 128), src_pair, dst_p, zeros128)
    t2 = _tc_mid(agg1, t1, dinv, b1.reshape(1, -1), g1.reshape(1, -1),
                 be1.reshape(1, -1), W2, split_out=True)
    agg2 = _sc_aggregate(t2.reshape(2 * N, 128), src_pair, dst_p, zeros128)
    t3 = _tc_mid(agg2, t2, dinv, b2.reshape(1, -1), g2.reshape(1, -1),
                 be2.reshape(1, -1), W3, split_out=False)
    agg3 = _sc_aggregate_full(t3, src_p, dst_p, zeros128)
    z = _tc_z(agg3, t3, dinv, b3.reshape(1, -1))

    dz = _sc_pair_gather(z, eli3)
    dz1 = dz[0]
    dz2 = dz[1]
    logits = _tc_decoder(dz1, dz2, dW1[:128], dW1[128:], db1.reshape(1, -1),
                         dW2, db2.reshape(1, -1), dW3, db3.reshape(1, -1),
                         dW4, db4.reshape(1, -1))
    return logits[:P]
